# baseline (speedup 1.0000x reference)
import jax
import jax.numpy as jnp
from jax import lax
from jax.experimental import pallas as pl
from jax.experimental.pallas import tpu as pltpu
from jax.experimental.pallas import tpu_sc as plsc

N = 2097152
N_CLASSES = 32
NC = 2
NS = 16
NW = NC * NS
PER_W = N // NW
CN = 256
NCHUNKS = PER_W // CN
NBUF = 8
LANES = 16
UNROLL = 4


def _body(in_hbm, out_hbm, *scr):
    in_vs = scr[0:NBUF]
    bin_vs = scr[NBUF:2 * NBUF]
    out_vs = scr[2 * NBUF:3 * NBUF]
    isems = scr[3 * NBUF:4 * NBUF]
    osems = scr[4 * NBUF:5 * NBUF]

    wid = lax.axis_index("s") * NC + lax.axis_index("c")
    base = wid * PER_W
    lane = lax.iota(jnp.int32, LANES)
    ones = jnp.full((LANES,), 1.0, jnp.float32)
    zeros = jnp.zeros((LANES,), jnp.float32)

    def zero(i, c):
        for b in range(NBUF):
            for r in range(N_CLASSES):
                out_vs[b][r, pl.ds(i * LANES, LANES)] = zeros
        return c

    lax.fori_loop(0, CN // LANES, zero, 0)

    def start_in(k, b):
        kk = lax.rem(k, NCHUNKS) if not isinstance(k, int) else k % NCHUNKS
        pltpu.async_copy(in_hbm.at[pl.ds(base + kk * CN, CN)], in_vs[b], isems[b])

    def wait_in(b):
        pltpu.make_async_copy(in_hbm.at[pl.ds(base, CN)], in_vs[b], isems[b]).wait()

    def wait_out(b):
        pltpu.make_async_copy(
            out_vs[b], out_hbm.at[:, pl.ds(base, CN)], osems[b]
        ).wait()

    def start_out(k, b):
        pltpu.async_copy(out_vs[b], out_hbm.at[:, pl.ds(base + k * CN, CN)], osems[b])

    def fill(b):
        def body(j, c):
            for u in range(UNROLL):
                o = j * (LANES * UNROLL) + u * LANES
                x = in_vs[b][pl.ds(o, LANES)]
                idx = (x * 32.0).astype(jnp.int32)
                col = o + lane
                bin_vs[b][pl.ds(o, LANES)] = idx
                plsc.store_scatter(out_vs[b], [idx, col], ones)
            return c

        lax.fori_loop(0, CN // (LANES * UNROLL), body, 0)

    def clear(b):
        def body(j, c):
            for u in range(UNROLL):
                o = j * (LANES * UNROLL) + u * LANES
                idx = bin_vs[b][pl.ds(o, LANES)]
                col = o + lane
                plsc.store_scatter(out_vs[b], [idx, col], zeros)
            return c

        lax.fori_loop(0, CN // (LANES * UNROLL), body, 0)

    for b in range(NBUF):
        start_in(b, b)

    for b in range(NBUF):
        wait_in(b)
        fill(b)
        start_out(b, b)
        start_in(b + NBUF, b)

    def group(g, c):
        for b in range(NBUF):
            k = NBUF * g + b
            wait_in(b)
            wait_out(b)
            clear(b)
            fill(b)
            start_out(k, b)
            start_in(k + NBUF, b)
        return c

    lax.fori_loop(1, NCHUNKS // NBUF, group, 0)

    for b in range(NBUF):
        wait_in(b)
        wait_out(b)


def kernel(inputs):
    mesh = plsc.VectorSubcoreMesh(core_axis_name="c", subcore_axis_name="s")
    f = pl.kernel(
        _body,
        mesh=mesh,
        out_type=jax.ShapeDtypeStruct((N_CLASSES, N), jnp.float32),
        compiler_params=pltpu.CompilerParams(
            needs_layout_passes=False, use_tc_tiling_on_sc=True
        ),
        scratch_types=(
            [pltpu.VMEM((CN,), jnp.float32) for _ in range(NBUF)]
            + [pltpu.VMEM((CN,), jnp.int32) for _ in range(NBUF)]
            + [pltpu.VMEM((N_CLASSES, CN), jnp.float32) for _ in range(NBUF)]
            + [pltpu.SemaphoreType.DMA for _ in range(2 * NBUF)]
        ),
    )
    return jnp.transpose(f(inputs))


# speedup vs baseline: 1.0354x; 1.0354x over previous
import jax
import jax.numpy as jnp
from jax import lax
from jax.experimental import pallas as pl
from jax.experimental.pallas import tpu as pltpu
from jax.experimental.pallas import tpu_sc as plsc

N = 2097152
N_CLASSES = 32
NC = 2
NS = 16
NW = NC * NS
PER_W = N // NW
CN = 512
NCHUNKS = PER_W // CN
NBUF = 4
LANES = 16
UNROLL = 4


def _body(in_hbm, out_hbm, *scr):
    in_vs = scr[0:NBUF]
    bin_vs = scr[NBUF:2 * NBUF]
    out_vs = scr[2 * NBUF:3 * NBUF]
    isems = scr[3 * NBUF:4 * NBUF]
    osems = scr[4 * NBUF:5 * NBUF]

    wid = lax.axis_index("s") * NC + lax.axis_index("c")
    base = wid * PER_W
    lane = lax.iota(jnp.int32, LANES)
    ones = jnp.full((LANES,), 1.0, jnp.float32)
    zeros = jnp.zeros((LANES,), jnp.float32)

    def zero(b):
        def body(i, c):
            for r in range(N_CLASSES):
                out_vs[b][r, pl.ds(i * LANES, LANES)] = zeros
            return c

        lax.fori_loop(0, CN // LANES, body, 0)

    def start_in(k, b):
        kk = lax.rem(k, NCHUNKS) if not isinstance(k, int) else k % NCHUNKS
        pltpu.async_copy(in_hbm.at[pl.ds(base + kk * CN, CN)], in_vs[b], isems[b])

    def wait_in(b):
        pltpu.make_async_copy(in_hbm.at[pl.ds(base, CN)], in_vs[b], isems[b]).wait()

    def wait_out(b):
        pltpu.make_async_copy(
            out_vs[b], out_hbm.at[:, pl.ds(base, CN)], osems[b]
        ).wait()

    def start_out(k, b):
        pltpu.async_copy(out_vs[b], out_hbm.at[:, pl.ds(base + k * CN, CN)], osems[b])

    def fill(b):
        def body(j, c):
            for u in range(UNROLL):
                o = j * (LANES * UNROLL) + u * LANES
                x = in_vs[b][pl.ds(o, LANES)]
                idx = (x * 32.0).astype(jnp.int32)
                col = o + lane
                bin_vs[b][pl.ds(o, LANES)] = idx
                plsc.store_scatter(out_vs[b], [idx, col], ones)
            return c

        lax.fori_loop(0, CN // (LANES * UNROLL), body, 0)

    def clear(b):
        def body(j, c):
            for u in range(UNROLL):
                o = j * (LANES * UNROLL) + u * LANES
                idx = bin_vs[b][pl.ds(o, LANES)]
                col = o + lane
                plsc.store_scatter(out_vs[b], [idx, col], zeros)
            return c

        lax.fori_loop(0, CN // (LANES * UNROLL), body, 0)

    for b in range(NBUF):
        start_in(b, b)

    for b in range(NBUF):
        zero(b)
        wait_in(b)
        fill(b)
        start_out(b, b)
        start_in(b + NBUF, b)

    def group(g, c):
        for b in range(NBUF):
            k = NBUF * g + b
            wait_in(b)
            wait_out(b)
            clear(b)
            fill(b)
            start_out(k, b)
            start_in(k + NBUF, b)
        return c

    lax.fori_loop(1, NCHUNKS // NBUF, group, 0)

    for b in range(NBUF):
        wait_in(b)
        wait_out(b)


def kernel(inputs):
    mesh = plsc.VectorSubcoreMesh(core_axis_name="c", subcore_axis_name="s")
    f = pl.kernel(
        _body,
        mesh=mesh,
        out_type=jax.ShapeDtypeStruct((N_CLASSES, N), jnp.float32),
        compiler_params=pltpu.CompilerParams(
            needs_layout_passes=False, use_tc_tiling_on_sc=True
        ),
        scratch_types=(
            [pltpu.VMEM((CN,), jnp.float32) for _ in range(NBUF)]
            + [pltpu.VMEM((CN,), jnp.int32) for _ in range(NBUF)]
            + [pltpu.VMEM((N_CLASSES, CN), jnp.float32) for _ in range(NBUF)]
            + [pltpu.SemaphoreType.DMA for _ in range(2 * NBUF)]
        ),
    )
    return jnp.transpose(f(inputs))
